# baseline (speedup 1.0000x reference)
import functools
import math

import jax
import jax.numpy as jnp
from jax.experimental import pallas as pl
from jax.experimental.pallas import tpu as pltpu

B, D, N = 1024, 1024, 8192
TEMP = 0.05
TN = 1024
NT = N // TN
LOG2E = math.log2(math.e)
LN2 = math.log(2.0)


def _fused_kernel(x_ref, f_ref, t_ref, out_ref, xs_ref, s_ref, ta_ref):
    j = pl.program_id(0)

    @pl.when(j == 0)
    def _init():
        xf = x_ref[...]
        norm = jnp.sqrt(jnp.sum(xf * xf, axis=1, keepdims=True))
        scale = LOG2E / (jnp.maximum(norm, 1e-12) * TEMP)
        xs_ref[...] = (xf * scale).astype(jnp.bfloat16)
        s_ref[...] = jnp.zeros_like(s_ref)
        ta_ref[...] = jnp.zeros_like(ta_ref)

    logits2 = jax.lax.dot_general(
        xs_ref[...], f_ref[...].astype(jnp.bfloat16),
        (((1,), (1,)), ((), ())),
        preferred_element_type=jnp.float32,
    )
    s_ref[...] += jnp.sum(jnp.exp2(logits2), axis=1, keepdims=True)
    col = jax.lax.broadcasted_iota(jnp.int32, (B, TN), 1) + j * TN
    ta_ref[...] += jnp.sum(
        jnp.where(col == t_ref[...], logits2, 0.0), axis=1, keepdims=True
    )

    @pl.when(j == NT - 1)
    def _fin():
        out_ref[0, 0] = jnp.sum(jnp.log(s_ref[...]) - ta_ref[...] * LN2) / B


def kernel(inputs, inputs_ema, targets, features):
    del inputs_ema
    tgt = targets.astype(jnp.int32).reshape(B, 1)
    return _fused_loss_full(inputs, features, tgt)


@jax.jit
def _fused_loss_full(inputs, features, tgt):
    out = pl.pallas_call(
        _fused_kernel,
        grid=(NT,),
        in_specs=[
            pl.BlockSpec((B, D), lambda j: (0, 0)),
            pl.BlockSpec((TN, D), lambda j: (j, 0)),
            pl.BlockSpec((B, 1), lambda j: (0, 0)),
        ],
        out_specs=pl.BlockSpec(memory_space=pltpu.SMEM),
        out_shape=jax.ShapeDtypeStruct((1, 1), jnp.float32),
        scratch_shapes=[
            pltpu.VMEM((B, D), jnp.bfloat16),
            pltpu.VMEM((B, 1), jnp.float32),
            pltpu.VMEM((B, 1), jnp.float32),
        ],
        compiler_params=pltpu.CompilerParams(
            dimension_semantics=("arbitrary",),
        ),
    )(inputs, features, tgt)
    return out[0, 0]
